# baseline (speedup 1.0000x reference)
import functools

import jax
import jax.numpy as jnp
from jax.experimental import pallas as pl
from jax.experimental.pallas import tpu as pltpu
from jax.sharding import Mesh, PartitionSpec as P

_VMEM_LIMIT = 48 * 1024 * 1024
_ROW_TILE = 256


def _prep_kernel(adj_ref, x_ref, w0_ref, abf_ref, d_ref, u0_ref):
    a = adj_ref[...]
    abf_ref[...] = a.astype(jnp.bfloat16)
    d = jax.lax.rsqrt(jnp.sum(a, axis=1, keepdims=True) + 1.0)
    d_ref[...] = d
    z = jnp.dot(x_ref[...].astype(jnp.bfloat16), w0_ref[...],
                preferred_element_type=jnp.float32)
    u0_ref[...] = (d * z).astype(jnp.bfloat16)


def _mid_kernel(abf_ref, uf_ref, ul_ref, d_ref, b_ref, w_ref, un_ref):
    agg = jnp.dot(abf_ref[...], uf_ref[...],
                  preferred_element_type=jnp.float32)
    u_strip = ul_ref[...].astype(jnp.float32)
    d = d_ref[...]
    out = d * (agg + u_strip) + b_ref[...]
    h = jnp.maximum(out, 0.0).astype(jnp.bfloat16)
    z = jnp.dot(h, w_ref[...], preferred_element_type=jnp.float32)
    un_ref[...] = (d * z).astype(jnp.bfloat16)


def _final_kernel(abf_ref, uf_ref, ul_ref, d_ref, b_ref, o_ref):
    agg = jnp.dot(abf_ref[...], uf_ref[...],
                  preferred_element_type=jnp.float32)
    u_strip = ul_ref[...].astype(jnp.float32)
    o_ref[...] = d_ref[...] * (agg + u_strip) + b_ref[...]


def _gcn_shard(x_l, adj_l, w0, b0, w1, b1, w2, b2, *, n, ndev):
    nloc = n // ndev
    tm = _ROW_TILE
    spc = nloc // tm
    f_h = w1.shape[0]
    f_out = w2.shape[1]

    cparams = pltpu.CompilerParams(
        dimension_semantics=("arbitrary",),
        vmem_limit_bytes=_VMEM_LIMIT,
    )
    strip = lambda s: (s, 0)
    const = lambda s: (0, 0)

    abf, d_is, u0_l = pl.pallas_call(
        _prep_kernel,
        grid=(spc,),
        in_specs=[
            pl.BlockSpec((tm, n), strip),
            pl.BlockSpec((tm, x_l.shape[1]), strip),
            pl.BlockSpec((x_l.shape[1], f_h), const),
        ],
        out_specs=[
            pl.BlockSpec((tm, n), strip),
            pl.BlockSpec((tm, 1), strip),
            pl.BlockSpec((tm, f_h), strip),
        ],
        out_shape=[
            jax.ShapeDtypeStruct((nloc, n), jnp.bfloat16),
            jax.ShapeDtypeStruct((nloc, 1), jnp.float32),
            jax.ShapeDtypeStruct((nloc, f_h), jnp.bfloat16),
        ],
        compiler_params=cparams,
    )(adj_l, x_l, w0)

    def gather(u_l):
        if ndev == 1:
            return u_l
        return jax.lax.all_gather(u_l, "i", axis=0, tiled=True)

    def mid(u_full, u_l, b, w):
        f_cur = u_full.shape[1]
        f_next = w.shape[1]
        return pl.pallas_call(
            _mid_kernel,
            grid=(spc,),
            in_specs=[
                pl.BlockSpec((tm, n), strip),
                pl.BlockSpec((n, f_cur), const),
                pl.BlockSpec((tm, f_cur), strip),
                pl.BlockSpec((tm, 1), strip),
                pl.BlockSpec((1, f_cur), const),
                pl.BlockSpec((f_cur, f_next), const),
            ],
            out_specs=pl.BlockSpec((tm, f_next), strip),
            out_shape=jax.ShapeDtypeStruct((nloc, f_next), jnp.bfloat16),
            compiler_params=cparams,
        )(abf, u_full, u_l, d_is, b, w)

    u1_l = mid(gather(u0_l), u0_l, b0, w1)
    u2_l = mid(gather(u1_l), u1_l, b1, w2)
    u2 = gather(u2_l)

    out_l = pl.pallas_call(
        _final_kernel,
        grid=(spc,),
        in_specs=[
            pl.BlockSpec((tm, n), strip),
            pl.BlockSpec((n, f_out), const),
            pl.BlockSpec((tm, f_out), strip),
            pl.BlockSpec((tm, 1), strip),
            pl.BlockSpec((1, f_out), const),
        ],
        out_specs=pl.BlockSpec((tm, f_out), strip),
        out_shape=jax.ShapeDtypeStruct((nloc, f_out), jnp.float32),
        compiler_params=cparams,
    )(abf, u2, u2_l, d_is, b2)
    return out_l


def kernel(x, adj, w_0, b_0, w_1, b_1, w_2, b_2):
    n = x.shape[0]

    w0 = w_0.astype(jnp.bfloat16)
    w1 = w_1.astype(jnp.bfloat16)
    w2 = w_2.astype(jnp.bfloat16)
    b0 = b_0.reshape(1, -1).astype(jnp.float32)
    b1 = b_1.reshape(1, -1).astype(jnp.float32)
    b2 = b_2.reshape(1, -1).astype(jnp.float32)

    devs = jax.devices()
    ndev = 2 if (len(devs) >= 2 and n % (2 * _ROW_TILE) == 0) else 1
    fwd = functools.partial(_gcn_shard, n=n, ndev=ndev)

    if ndev == 1:
        return fwd(x, adj, w0, b0, w1, b1, w2, b2)

    mesh = Mesh(devs[:ndev], ("i",))
    shard = P("i", None)
    rep = P(None, None)
    return jax.shard_map(
        fwd,
        mesh=mesh,
        in_specs=(shard, shard, rep, rep, rep, rep, rep, rep),
        out_specs=shard,
        check_vma=False,
    )(x, adj, w0, b0, w1, b1, w2, b2)


# speedup vs baseline: 9.9323x; 9.9323x over previous
import functools

import jax
import jax.numpy as jnp
from jax.experimental import pallas as pl
from jax.experimental.pallas import tpu as pltpu

_VMEM_LIMIT = 50 * 1024 * 1024
_ROW_TILE = 256


def _gcn_kernel(adj_ref, x_ref, w_ref, b_ref, o_ref,
                abf, ua, ub, dsc, *, nstrips, f_out):
    s = pl.program_id(0)
    m = pl.program_id(1)
    tm = o_ref.shape[0]
    r0 = pl.multiple_of(m * tm, tm)
    rows = pl.ds(r0, tm)

    @pl.when(s == 0)
    def _prep():
        a = adj_ref[...]
        abf[rows, :] = a.astype(jnp.bfloat16)
        d = jax.lax.rsqrt(jnp.sum(a, axis=1, keepdims=True) + 1.0)
        dsc[rows, :] = d
        z = jnp.dot(x_ref[...].astype(jnp.bfloat16), w_ref[0],
                    preferred_element_type=jnp.float32)
        ua[rows, :] = (d * z).astype(jnp.bfloat16)

    def propagate(u_cur, width):
        agg = jnp.dot(abf[rows, :], u_cur[:, :width],
                      preferred_element_type=jnp.float32)
        u_strip = u_cur[rows, :width].astype(jnp.float32)
        d = dsc[rows, :]
        return d, d * (agg + u_strip) + b_ref[0][:, :width]

    @pl.when(s == 1)
    def _layer0():
        d, out = propagate(ua, ua.shape[1])
        h = jnp.maximum(out, 0.0).astype(jnp.bfloat16)
        z = jnp.dot(h, w_ref[0], preferred_element_type=jnp.float32)
        ub[rows, :] = (d * z).astype(jnp.bfloat16)

    @pl.when(s == 2)
    def _layer1():
        d, out = propagate(ub, ub.shape[1])
        h = jnp.maximum(out, 0.0).astype(jnp.bfloat16)
        z = jnp.dot(h, w_ref[0][:, :f_out],
                    preferred_element_type=jnp.float32)
        ua[rows, :f_out] = (d * z).astype(jnp.bfloat16)

    @pl.when(s == 3)
    def _layer2():
        _, out = propagate(ua, f_out)
        o_ref[...] = out


def kernel(x, adj, w_0, b_0, w_1, b_1, w_2, b_2):
    n, f_in = x.shape
    f_h = w_1.shape[0]
    f_out = w_2.shape[1]
    tm = _ROW_TILE
    nstrips = n // tm
    num_stages = 4

    w_stack = jnp.zeros((num_stages, f_in, f_h), jnp.bfloat16)
    w_stack = w_stack.at[0].set(w_0.astype(jnp.bfloat16))
    w_stack = w_stack.at[1].set(w_1.astype(jnp.bfloat16))
    w_stack = w_stack.at[2, :, :f_out].set(w_2.astype(jnp.bfloat16))
    b_stack = jnp.zeros((num_stages, 1, f_h), jnp.float32)
    b_stack = b_stack.at[1, 0, :].set(b_0.astype(jnp.float32))
    b_stack = b_stack.at[2, 0, :].set(b_1.astype(jnp.float32))
    b_stack = b_stack.at[3, 0, :f_out].set(b_2.astype(jnp.float32))

    last = nstrips - 1
    adj_idx = lambda s, m: (jax.lax.select(s == 0, m, last), 0)
    out_idx = lambda s, m: (jax.lax.select(s == num_stages - 1, m, 0), 0)
    wb_idx = lambda s, m: (jax.lax.min(s, num_stages - 2), 0, 0)

    return pl.pallas_call(
        functools.partial(_gcn_kernel, nstrips=nstrips, f_out=f_out),
        grid=(num_stages, nstrips),
        in_specs=[
            pl.BlockSpec((tm, n), adj_idx),
            pl.BlockSpec((tm, f_in), adj_idx),
            pl.BlockSpec((1, f_in, f_h), wb_idx),
            pl.BlockSpec((1, 1, f_h), lambda s, m: (s, 0, 0)),
        ],
        out_specs=pl.BlockSpec((tm, f_out), out_idx),
        out_shape=jax.ShapeDtypeStruct((n, f_out), jnp.float32),
        scratch_shapes=[
            pltpu.VMEM((n, n), jnp.bfloat16),
            pltpu.VMEM((n, f_h), jnp.bfloat16),
            pltpu.VMEM((n, f_h), jnp.bfloat16),
            pltpu.VMEM((n, 1), jnp.float32),
        ],
        compiler_params=pltpu.CompilerParams(
            dimension_semantics=("arbitrary", "arbitrary"),
            vmem_limit_bytes=_VMEM_LIMIT,
        ),
    )(adj, x, w_stack, b_stack)


# speedup vs baseline: 11.3635x; 1.1441x over previous
import functools

import jax
import jax.numpy as jnp
from jax.experimental import pallas as pl
from jax.experimental.pallas import tpu as pltpu

_VMEM_LIMIT = 50 * 1024 * 1024
_ROW_TILE = 512


def _gcn_kernel(adj_ref, x_ref, w_ref, b_ref, o_ref,
                abf, ua, ub, dsc, *, nstrips, f_out):
    s = pl.program_id(0)
    m = pl.program_id(1)
    tm = o_ref.shape[0]
    r0 = pl.multiple_of(m * tm, tm)
    rows = pl.ds(r0, tm)

    @pl.when(s == 0)
    def _prep():
        a = adj_ref[...]
        col = jax.lax.broadcasted_iota(jnp.int32, a.shape, 1)
        row = jax.lax.broadcasted_iota(jnp.int32, a.shape, 0) + r0
        abf[rows, :] = jnp.where(col == row, a + 1.0, a).astype(jnp.bfloat16)
        d = jax.lax.rsqrt(jnp.sum(a, axis=1, keepdims=True) + 1.0)
        dsc[rows, :] = d
        z = jnp.dot(x_ref[...].astype(jnp.bfloat16), w_ref[0],
                    preferred_element_type=jnp.float32)
        ua[rows, :] = (d * z).astype(jnp.bfloat16)

    def propagate(u_cur, width):
        agg = jnp.dot(abf[rows, :], u_cur[:, :width],
                      preferred_element_type=jnp.float32)
        d = dsc[rows, :]
        return d, d * agg + b_ref[0][:, :width]

    @pl.when(s == 1)
    def _layer0():
        d, out = propagate(ua, ua.shape[1])
        h = jnp.maximum(out, 0.0).astype(jnp.bfloat16)
        z = jnp.dot(h, w_ref[0], preferred_element_type=jnp.float32)
        ub[rows, :] = (d * z).astype(jnp.bfloat16)

    @pl.when(s == 2)
    def _layer1():
        d, out = propagate(ub, ub.shape[1])
        h = jnp.maximum(out, 0.0).astype(jnp.bfloat16)
        z = jnp.dot(h, w_ref[0][:, :f_out],
                    preferred_element_type=jnp.float32)
        ua[rows, :f_out] = (d * z).astype(jnp.bfloat16)

    @pl.when(s == 3)
    def _layer2():
        _, out = propagate(ua, f_out)
        o_ref[...] = out


def kernel(x, adj, w_0, b_0, w_1, b_1, w_2, b_2):
    n, f_in = x.shape
    f_h = w_1.shape[0]
    f_out = w_2.shape[1]
    tm = _ROW_TILE
    nstrips = n // tm
    num_stages = 4

    w_stack = jnp.zeros((num_stages, f_in, f_h), jnp.bfloat16)
    w_stack = w_stack.at[0].set(w_0.astype(jnp.bfloat16))
    w_stack = w_stack.at[1].set(w_1.astype(jnp.bfloat16))
    w_stack = w_stack.at[2, :, :f_out].set(w_2.astype(jnp.bfloat16))
    b_stack = jnp.zeros((num_stages, 1, f_h), jnp.float32)
    b_stack = b_stack.at[1, 0, :].set(b_0.astype(jnp.float32))
    b_stack = b_stack.at[2, 0, :].set(b_1.astype(jnp.float32))
    b_stack = b_stack.at[3, 0, :f_out].set(b_2.astype(jnp.float32))

    last = nstrips - 1
    adj_idx = lambda s, m: (jax.lax.select(s == 0, m, last), 0)
    out_idx = lambda s, m: (jax.lax.select(s == num_stages - 1, m, 0), 0)
    wb_idx = lambda s, m: (jax.lax.min(s, num_stages - 2), 0, 0)

    return pl.pallas_call(
        functools.partial(_gcn_kernel, nstrips=nstrips, f_out=f_out),
        grid=(num_stages, nstrips),
        in_specs=[
            pl.BlockSpec((tm, n), adj_idx),
            pl.BlockSpec((tm, f_in), adj_idx),
            pl.BlockSpec((1, f_in, f_h), wb_idx),
            pl.BlockSpec((1, 1, f_h), lambda s, m: (s, 0, 0)),
        ],
        out_specs=pl.BlockSpec((tm, f_out), out_idx),
        out_shape=jax.ShapeDtypeStruct((n, f_out), jnp.float32),
        scratch_shapes=[
            pltpu.VMEM((n, n), jnp.bfloat16),
            pltpu.VMEM((n, f_h), jnp.bfloat16),
            pltpu.VMEM((n, f_h), jnp.bfloat16),
            pltpu.VMEM((n, 1), jnp.float32),
        ],
        compiler_params=pltpu.CompilerParams(
            dimension_semantics=("arbitrary", "arbitrary"),
            vmem_limit_bytes=_VMEM_LIMIT,
        ),
    )(adj, x, w_stack, b_stack)
